# baseline (speedup 1.0000x reference)
import functools

import jax
import jax.numpy as jnp
from jax import lax
from jax.experimental import pallas as pl
from jax.experimental.pallas import tpu as pltpu
from jax.experimental.pallas import tpu_sc as plsc

BATCH = 16384
D = 256
NC = 2
NS = 16
NW = NC * NS
BPW = BATCH // NW
CH = 64
NCHUNK = BPW // CH
NBUF = 3

_mesh = plsc.VectorSubcoreMesh(core_axis_name="c", subcore_axis_name="s")


@functools.partial(
    pl.kernel,
    mesh=_mesh,
    out_type=jax.ShapeDtypeStruct((BATCH, 2 * D), jnp.float32),
    scratch_types=[
        pltpu.VMEM((NCHUNK, CH), jnp.int32),
        pltpu.VMEM((NCHUNK, CH), jnp.int32),
    ] + [pltpu.VMEM((CH, 2 * D), jnp.float32) for _ in range(NBUF)]
      + [pltpu.SemaphoreType.DMA for _ in range(2 * NBUF)]
      + [pltpu.SemaphoreType.DMA],
)
def _gather_kernel(oidx_hbm, cidx_hbm, otab_hbm, ctab_hbm, out_hbm,
                   oidx_v, cidx_v, b0, b1, b2, g0, g1, g2, s0, s1, s2, isem):
    bufs = (b0, b1, b2)
    gsems = (g0, g1, g2)
    osems = (s0, s1, s2)
    wid = lax.axis_index("s") * NC + lax.axis_index("c")
    base = wid * BPW
    icp0 = pltpu.async_copy(oidx_hbm.at[pl.ds(wid * NCHUNK, NCHUNK)], oidx_v, isem)
    icp1 = pltpu.async_copy(cidx_hbm.at[pl.ds(wid * NCHUNK, NCHUNK)], cidx_v, isem)
    icp0.wait()
    icp1.wait()
    gcp = [None] * NCHUNK
    ocp = [None] * NCHUNK
    for c in range(NCHUNK + 1):
        if c >= 1:
            cp = c - 1
            slot = cp % NBUF
            gcp[cp][0].wait()
            oo = pltpu.async_copy(
                bufs[slot].at[:, pl.ds(0, D)],
                out_hbm.at[pl.ds(base + cp * CH, CH), pl.ds(0, D)], osems[slot])
            gcp[cp][1].wait()
            oc = pltpu.async_copy(
                bufs[slot].at[:, pl.ds(D, D)],
                out_hbm.at[pl.ds(base + cp * CH, CH), pl.ds(D, D)], osems[slot])
            ocp[cp] = (oo, oc)
        if c < NCHUNK:
            slot = c % NBUF
            if c >= NBUF:
                ocp[c - NBUF][0].wait()
                ocp[c - NBUF][1].wait()
            go = pltpu.async_copy(otab_hbm.at[oidx_v.at[c]],
                                  bufs[slot].at[:, pl.ds(0, D)], gsems[slot])
            gc = pltpu.async_copy(ctab_hbm.at[cidx_v.at[c]],
                                  bufs[slot].at[:, pl.ds(D, D)], gsems[slot])
            gcp[c] = (go, gc)
    for c in range(NCHUNK - NBUF, NCHUNK):
        ocp[c][0].wait()
        ocp[c][1].wait()


def kernel(input, obj_table, color_table):
    obj_idx = input[:, 0].reshape(BATCH // CH, CH)
    color_idx = input[:, 1].reshape(BATCH // CH, CH)
    return _gather_kernel(obj_idx, color_idx, obj_table, color_table)


# speedup vs baseline: 1.0320x; 1.0320x over previous
import functools

import jax
import jax.numpy as jnp
from jax import lax
from jax.experimental import pallas as pl
from jax.experimental.pallas import tpu as pltpu
from jax.experimental.pallas import tpu_sc as plsc

BATCH = 16384
D = 256
NC = 2
NS = 16
NW = NC * NS
BPW = BATCH // NW
CH = 64
NCHUNK = BPW // CH
NBUF = 3

_mesh = plsc.VectorSubcoreMesh(core_axis_name="c", subcore_axis_name="s")


@functools.partial(
    pl.kernel,
    mesh=_mesh,
    out_type=jax.ShapeDtypeStruct((BATCH, 2 * D), jnp.float32),
    scratch_types=[
        pltpu.VMEM((NCHUNK, CH), jnp.int32),
        pltpu.VMEM((NCHUNK, CH), jnp.int32),
    ] + [pltpu.VMEM((CH, 2 * D), jnp.float32) for _ in range(NBUF)]
      + [pltpu.SemaphoreType.DMA for _ in range(2 * NBUF)]
      + [pltpu.SemaphoreType.DMA],
)
def _gather_kernel(oidx_hbm, cidx_hbm, otab_hbm, ctab_hbm, out_hbm,
                   oidx_v, cidx_v, b0, b1, b2, g0, g1, g2, s0, s1, s2, isem):
    bufs = (b0, b1, b2)
    gsems = (g0, g1, g2)
    osems = (s0, s1, s2)
    wid = lax.axis_index("s") * NC + lax.axis_index("c")
    base = wid * BPW
    icp0 = pltpu.async_copy(oidx_hbm.at[pl.ds(wid * NCHUNK, NCHUNK)], oidx_v, isem)
    icp1 = pltpu.async_copy(cidx_hbm.at[pl.ds(wid * NCHUNK, NCHUNK)], cidx_v, isem)
    icp0.wait()
    icp1.wait()
    gcp = [None] * NCHUNK
    ocp = [None] * NCHUNK
    for c in range(NCHUNK + 1):
        if c < NCHUNK:
            slot = c % NBUF
            if c >= NBUF:
                ocp[c - NBUF][0].wait()
                ocp[c - NBUF][1].wait()
            go = pltpu.async_copy(otab_hbm.at[oidx_v.at[c]],
                                  bufs[slot].at[:, pl.ds(0, D)], gsems[slot])
            gc = pltpu.async_copy(ctab_hbm.at[cidx_v.at[c]],
                                  bufs[slot].at[:, pl.ds(D, D)], gsems[slot])
            gcp[c] = (go, gc)
        if c >= 1:
            cp = c - 1
            slot = cp % NBUF
            gcp[cp][0].wait()
            oo = pltpu.async_copy(
                bufs[slot].at[:, pl.ds(0, D)],
                out_hbm.at[pl.ds(base + cp * CH, CH), pl.ds(0, D)], osems[slot])
            gcp[cp][1].wait()
            oc = pltpu.async_copy(
                bufs[slot].at[:, pl.ds(D, D)],
                out_hbm.at[pl.ds(base + cp * CH, CH), pl.ds(D, D)], osems[slot])
            ocp[cp] = (oo, oc)
    for c in range(NCHUNK - NBUF, NCHUNK):
        ocp[c][0].wait()
        ocp[c][1].wait()


def kernel(input, obj_table, color_table):
    obj_idx = input[:, 0].reshape(BATCH // CH, CH)
    color_idx = input[:, 1].reshape(BATCH // CH, CH)
    return _gather_kernel(obj_idx, color_idx, obj_table, color_table)
